# baseline (speedup 1.0000x reference)
import functools

import jax
import jax.numpy as jnp
from jax import lax
from jax.experimental import pallas as pl
from jax.experimental.pallas import tpu as pltpu
from jax.experimental.pallas import tpu_sc as plsc

N_LATENT = 32
N_OC = 16
B = 512
R = 4096
LANES = 16
N_WORKERS = 32

KO = N_LATENT * N_OC
KPW = KO // N_WORKERS


def _sc_gather_lanes(idx, wt):
    n_regions = wt.shape[1]
    half = n_regions // 2
    wt2 = wt.reshape(KO * 2, half)
    mesh = plsc.VectorSubcoreMesh(core_axis_name="c", subcore_axis_name="s")

    @functools.partial(
        pl.kernel,
        mesh=mesh,
        out_type=jax.ShapeDtypeStruct((KO, R), jnp.float32),
        scratch_types=[
            pltpu.VMEM((R,), jnp.int32),
            pltpu.VMEM((half,), jnp.float32),
            pltpu.VMEM((half,), jnp.float32),
            pltpu.VMEM((R,), jnp.float32),
            pltpu.VMEM((R,), jnp.float32),
            pltpu.SemaphoreType.DMA,
            pltpu.SemaphoreType.DMA,
            pltpu.SemaphoreType.DMA,
            pltpu.SemaphoreType.DMA,
        ],
        compiler_params=pltpu.CompilerParams(
            use_tc_tiling_on_sc=True, needs_layout_passes=False
        ),
    )
    def gather_kernel(
        idx_hbm, wt_hbm, out_hbm, idx_v, h0, h1, oa, ob, s0, s1, so0, so1
    ):
        w = lax.axis_index("s") * 2 + lax.axis_index("c")
        pltpu.sync_copy(idx_hbm, idx_v)
        obufs = (oa, ob)
        osems = (so0, so1)
        wb = [None, None]
        for t in range(KPW):
            k = w * KPW + t
            c0 = pltpu.async_copy(wt_hbm.at[2 * k], h0, s0)
            c1 = pltpu.async_copy(wt_hbm.at[2 * k + 1], h1, s1)
            c0.wait()
            c1.wait()
            orow_v = obufs[t % 2]
            if wb[t % 2] is not None:
                wb[t % 2].wait()

            def body(j, _, orow_v=orow_v):
                base = j * (LANES * 2)
                for u in range(2):
                    sl = pl.ds(base + u * LANES, LANES)
                    iv = idx_v[sl]
                    m = iv < half
                    g0 = plsc.load_gather(h0, [jnp.minimum(iv, half - 1)])
                    g1 = plsc.load_gather(h1, [jnp.maximum(iv - half, 0)])
                    orow_v[sl] = jnp.where(m, g0, g1)
                return 0

            lax.fori_loop(0, R // (LANES * 2), body, 0)
            wb[t % 2] = pltpu.async_copy(orow_v, out_hbm.at[k], osems[t % 2])
        wb[0].wait()
        wb[1].wait()

    return gather_kernel(idx, wt2)


def _height_matmul(latent, g3):
    OB = 8
    NBR = 512

    def body(lat_ref, g_ref, out_ref):
        for oo in range(OB):
            out_ref[:, oo, :] = jnp.dot(
                lat_ref[...], g_ref[:, oo, :], preferred_element_type=jnp.float32
            )

    return pl.pallas_call(
        body,
        grid=(N_OC // OB, R // NBR),
        in_specs=[
            pl.BlockSpec((B, N_LATENT), lambda o, j: (0, 0)),
            pl.BlockSpec((N_LATENT, OB, NBR), lambda o, j: (0, o, j)),
        ],
        out_specs=pl.BlockSpec((B, OB, NBR), lambda o, j: (0, o, j)),
        out_shape=jax.ShapeDtypeStruct((B, N_OC, R), jnp.float32),
    )(latent, g3)


def _baseline_matmul_t(bwT, latent):
    NB = 4096
    n_regions = bwT.shape[1]

    def body(bw_ref, lat_ref, out_ref):
        out_ref[...] = lax.dot_general(
            bw_ref[...],
            lat_ref[...],
            dimension_numbers=(((0,), (1,)), ((), ())),
            preferred_element_type=jnp.float32,
        )

    return pl.pallas_call(
        body,
        grid=(pl.cdiv(n_regions, NB),),
        in_specs=[
            pl.BlockSpec((N_LATENT, NB), lambda i: (0, i)),
            pl.BlockSpec((B, N_LATENT), lambda i: (0, 0)),
        ],
        out_specs=pl.BlockSpec((NB, B), lambda i: (i, 0)),
        out_shape=jax.ShapeDtypeStruct((n_regions, B), jnp.float32),
    )(bwT, latent)


def kernel(latent, regions_oi, height_w, baseline_w):
    n_regions = height_w.shape[0]
    wt = jnp.transpose(height_w, (1, 2, 0)).reshape(KO, n_regions)
    g_t = _sc_gather_lanes(regions_oi, wt)
    g3 = g_t.reshape(N_LATENT, N_OC, R)
    dh_p = _height_matmul(latent, g3)
    delta_height = jnp.transpose(dh_p, (0, 2, 1))
    db_t = _baseline_matmul_t(baseline_w.T, latent)
    delta_baseline = db_t.T
    return delta_height, delta_baseline


# speedup vs baseline: 1.6063x; 1.6063x over previous
import functools

import jax
import jax.numpy as jnp
from jax import lax
from jax.experimental import pallas as pl
from jax.experimental.pallas import tpu as pltpu
from jax.experimental.pallas import tpu_sc as plsc

N_LATENT = 32
N_OC = 16
B = 512
R = 4096
LANES = 16
N_WORKERS = 32

KO = N_LATENT * N_OC
KPW = KO // N_WORKERS


def _sc_gather_lanes(idx, wt):
    n_regions = wt.shape[1]
    mesh = plsc.VectorSubcoreMesh(core_axis_name="c", subcore_axis_name="s")

    @functools.partial(
        pl.kernel,
        mesh=mesh,
        out_type=jax.ShapeDtypeStruct((KO, R), jnp.float32),
        scratch_types=[
            pltpu.VMEM((R,), jnp.int32),
            pltpu.VMEM((n_regions,), jnp.float32),
            pltpu.VMEM((R,), jnp.float32),
        ],
        compiler_params=pltpu.CompilerParams(
            use_tc_tiling_on_sc=True, needs_layout_passes=False
        ),
    )
    def gather_kernel(idx_hbm, wt_hbm, out_hbm, idx_v, row_v, orow_v):
        w = lax.axis_index("s") * 2 + lax.axis_index("c")
        pltpu.sync_copy(idx_hbm, idx_v)
        for t in range(KPW):
            k = w * KPW + t
            pltpu.sync_copy(wt_hbm.at[k], row_v)

            def body(j, _):
                iv = idx_v[pl.ds(j * LANES, LANES)]
                orow_v[pl.ds(j * LANES, LANES)] = plsc.load_gather(row_v, [iv])
                return 0

            lax.fori_loop(0, R // LANES, body, 0)
            pltpu.sync_copy(orow_v, out_hbm.at[k])

    return gather_kernel(idx, wt)


def _height_matmul(latent, g3):
    OB = 8
    NBR = 512

    def body(lat_ref, g_ref, out_ref):
        for oo in range(OB):
            out_ref[:, oo, :] = jnp.dot(
                lat_ref[...], g_ref[:, oo, :], preferred_element_type=jnp.float32
            )

    return pl.pallas_call(
        body,
        grid=(N_OC // OB, R // NBR),
        in_specs=[
            pl.BlockSpec((B, N_LATENT), lambda o, j: (0, 0)),
            pl.BlockSpec((N_LATENT, OB, NBR), lambda o, j: (0, o, j)),
        ],
        out_specs=pl.BlockSpec((B, OB, NBR), lambda o, j: (0, o, j)),
        out_shape=jax.ShapeDtypeStruct((B, N_OC, R), jnp.float32),
    )(latent, g3)


def _baseline_matmul_t(bwT, latent):
    NB = 4096
    n_regions = bwT.shape[1]

    def body(bw_ref, lat_ref, out_ref):
        out_ref[...] = lax.dot_general(
            bw_ref[...],
            lat_ref[...],
            dimension_numbers=(((0,), (1,)), ((), ())),
            preferred_element_type=jnp.float32,
        )

    return pl.pallas_call(
        body,
        grid=(pl.cdiv(n_regions, NB),),
        in_specs=[
            pl.BlockSpec((N_LATENT, NB), lambda i: (0, i)),
            pl.BlockSpec((B, N_LATENT), lambda i: (0, 0)),
        ],
        out_specs=pl.BlockSpec((NB, B), lambda i: (i, 0)),
        out_shape=jax.ShapeDtypeStruct((n_regions, B), jnp.float32),
    )(bwT, latent)


def kernel(latent, regions_oi, height_w, baseline_w):
    n_regions = height_w.shape[0]
    wt = jnp.transpose(height_w, (1, 2, 0)).reshape(KO, n_regions)
    g_t = _sc_gather_lanes(regions_oi, wt)
    g3 = g_t.reshape(N_LATENT, N_OC, R)
    dh_p = _height_matmul(latent, g3)
    delta_height = jnp.transpose(dh_p, (0, 2, 1))
    db_t = _baseline_matmul_t(baseline_w.T, latent)
    delta_baseline = db_t.T
    return delta_height, delta_baseline
